# baseline (speedup 1.0000x reference)
import functools

import jax
import jax.numpy as jnp
from jax import lax
from jax.experimental import pallas as pl
from jax.experimental.pallas import tpu as pltpu
from jax.experimental.pallas import tpu_sc as plsc

_DIM = 64
_MARGIN = 1.0
_C = 0.25
_BATCH = 16384
_NW = 32
_PER_W = _BATCH // _NW
_CHUNK = 128
_NCHUNK = _PER_W // _CHUNK
_GROUPS = _CHUNK // 16


def _sqrt16(x):
    i = lax.bitcast_convert_type(x, jnp.int32)
    y = lax.bitcast_convert_type(
        jnp.int32(0x5F3759DF) - lax.shift_right_arithmetic(i, 1), jnp.float32)
    for _ in range(3):
        y = y * (1.5 - 0.5 * x * y * y)
    return x * y


def _transe_kernel(ent_hbm, rel_hbm, idx_hbm, out_hbm,
                   idx_h, idx_r, idx_t, hbuf, rbuf, tbuf, posq, ovec,
                   sem_h, sem_r, sem_t):
    nc = 2
    wid = lax.axis_index("s") * nc + lax.axis_index("c")
    base = wid * _PER_W
    iota = lax.broadcasted_iota(jnp.int32, (16,), 0)

    acc_loss = jnp.zeros((16,), jnp.float32)
    acc_e = jnp.zeros((16,), jnp.float32)
    acc_r = jnp.zeros((16,), jnp.float32)

    for phase in range(2):
        for c in range(_NCHUNK):
            off = phase * 3 * _BATCH + base + c * _CHUNK
            pltpu.sync_copy(idx_hbm.at[pl.ds(off, _CHUNK)], idx_h)
            pltpu.sync_copy(idx_hbm.at[pl.ds(off + _BATCH, _CHUNK)], idx_r)
            pltpu.sync_copy(idx_hbm.at[pl.ds(off + 2 * _BATCH, _CHUNK)], idx_t)
            cp_h = pltpu.async_copy(ent_hbm.at[idx_h], hbuf, sem_h)
            cp_r = pltpu.async_copy(rel_hbm.at[idx_r], rbuf, sem_r)
            cp_t = pltpu.async_copy(ent_hbm.at[idx_t], tbuf, sem_t)
            cp_h.wait()
            cp_r.wait()
            cp_t.wait()

            def group_body(g, carry):
                acc_loss, acc_e, acc_r = carry
                rowv = g * 16 + iota

                def dim_body(j, dcarry):
                    sq, nh, nt, nr = dcarry
                    colv = jnp.bitwise_and(iota + j, _DIM - 1)
                    hv = plsc.load_gather(hbuf, [rowv, colv])
                    rv = plsc.load_gather(rbuf, [rowv, colv])
                    tv = plsc.load_gather(tbuf, [rowv, colv])
                    d = hv + rv - tv
                    return (sq + d * d, nh + hv * hv, nt + tv * tv,
                            nr + rv * rv)

                z = jnp.zeros((16,), jnp.float32)
                sq, nh, nt, nr = lax.fori_loop(
                    0, _DIM, dim_body, (z, z, z, z), unroll=4)

                acc_e = acc_e + jnp.maximum(nh - 1.0, 0.0) \
                              + jnp.maximum(nt - 1.0, 0.0)
                acc_r = acc_r + jnp.maximum(nr - 1.0, 0.0)
                qslot = c * _GROUPS + g
                if phase == 0:
                    posq[pl.ds(qslot * 16, 16)] = sq
                else:
                    pos = _sqrt16(posq[pl.ds(qslot * 16, 16)])
                    neg = _sqrt16(sq)
                    acc_loss = acc_loss + jnp.maximum(
                        pos - neg + _MARGIN, 0.0)
                return (acc_loss, acc_e, acc_r)

            acc_loss, acc_e, acc_r = lax.fori_loop(
                0, _GROUPS, group_body, (acc_loss, acc_e, acc_r))

    ovec[...] = (acc_loss * (1.0 / _BATCH)
                 + acc_e * (_C / (4.0 * _BATCH))
                 + acc_r * (_C / (2.0 * _BATCH)))
    pltpu.sync_copy(ovec, out_hbm.at[wid])


@jax.jit
def kernel(current_triples, corrupted_triples, ent_emb, rel_emb):
    idx = jnp.concatenate(
        [current_triples.T.reshape(-1), corrupted_triples.T.reshape(-1)])
    mesh = plsc.VectorSubcoreMesh(core_axis_name="c", subcore_axis_name="s")
    run = pl.kernel(
        _transe_kernel,
        out_type=jax.ShapeDtypeStruct((_NW, 16), jnp.float32),
        mesh=mesh,
        compiler_params=pltpu.CompilerParams(
            needs_layout_passes=False, use_tc_tiling_on_sc=False),
        scratch_types=[
            pltpu.VMEM((_CHUNK,), jnp.int32),
            pltpu.VMEM((_CHUNK,), jnp.int32),
            pltpu.VMEM((_CHUNK,), jnp.int32),
            pltpu.VMEM((_CHUNK, _DIM), jnp.float32),
            pltpu.VMEM((_CHUNK, _DIM), jnp.float32),
            pltpu.VMEM((_CHUNK, _DIM), jnp.float32),
            pltpu.VMEM((_PER_W,), jnp.float32),
            pltpu.VMEM((16,), jnp.float32),
            pltpu.SemaphoreType.DMA,
            pltpu.SemaphoreType.DMA,
            pltpu.SemaphoreType.DMA,
        ],
    )
    partials = run(ent_emb, rel_emb, idx)
    return jnp.sum(partials)
